# baseline (speedup 1.0000x reference)
import functools

import jax
import jax.numpy as jnp
from jax import lax
from jax.experimental import pallas as pl
from jax.experimental.pallas import tpu as pltpu
from jax.experimental.pallas import tpu_sc as plsc

_VOCAB = 1_000_000
_D = 64
_B = 16384

_NC = 2
_NS = 16
_NW = _NC * _NS
_BLK = 256
_FULL_BLKS = _VOCAB // _BLK
_TAIL_W = _VOCAB - _FULL_BLKS * _BLK
_BASE_BLKS = _FULL_BLKS // _NW
_EXTRA = _FULL_BLKS - _BASE_BLKS * _NW
_XCHUNK = 4096
_CAP = 256
_TRASH = _B
_RING = 4


def _i16(v):
    return jnp.full((16,), v, dtype=jnp.int32)


@functools.partial(
    pl.kernel,
    out_type=jax.ShapeDtypeStruct((_B + 16, 2 * _D), jnp.float32),
    mesh=plsc.VectorSubcoreMesh(core_axis_name="c", subcore_axis_name="s"),
    scratch_types=[
        pltpu.VMEM((_XCHUNK,), jnp.int32),
        pltpu.VMEM((_B,), jnp.int32),
        pltpu.VMEM((_B,), jnp.int32),
        pltpu.VMEM((_B + 64,), jnp.int32),
        pltpu.VMEM((2, _D, _BLK), jnp.float32),
        pltpu.VMEM((_CAP,), jnp.int32),
        pltpu.VMEM((_CAP,), jnp.int32),
        pltpu.VMEM((2, 64, 2 * _D), jnp.float32),
        pltpu.VMEM((2, 64), jnp.int32),
        pltpu.VMEM((_D, 16), jnp.float32),
        pltpu.SemaphoreType.DMA,
        pltpu.SemaphoreType.DMA,
        pltpu.SemaphoreType.DMA,
    ],
    compiler_params=pltpu.CompilerParams(needs_layout_passes=False),
)
def _decoder_kernel(
    x_hbm, tt_hbm, tail_hbm, out_hbm,
    xc, lx, lp, lb, bufs, blx, blp, stage, sidx, colb,
    sem0, sem1, sem_out,
):
    wid = lax.axis_index("s") * _NC + lax.axis_index("c")
    nblk = _BASE_BLKS + jnp.where(wid < _EXTRA, 1, 0)
    b0 = _BASE_BLKS * wid + jnp.minimum(wid, _EXTRA)
    lo = b0 * _BLK
    is_last = wid == _NW - 1
    hi = jnp.where(is_last, _VOCAB, lo + nblk * _BLK)
    lanes = lax.iota(jnp.int32, 16)

    def chunk_scan(c, n):
        pltpu.sync_copy(
            x_hbm.at[pl.ds(pl.multiple_of(c * _XCHUNK, 1024), _XCHUNK)], xc
        )

        def grp(g, tot):
            for u in range(4):
                xg = xc[pl.ds(g * 64 + u * 16, 16)]
                m = (xg >= lo) & (xg < hi)
                cs = plsc.cumsum(jnp.where(m, 1, 0))
                slots = cs + tot - 1
                off = xg - lo
                plsc.store_scatter(lx, [slots], off, mask=m)
                plsc.store_scatter(lb, [slots], off >> 8, mask=m)
                pos = c * _XCHUNK + g * 64 + u * 16 + lanes
                plsc.store_scatter(lp, [slots], pos, mask=m)
                tot = tot + cs.at[_i16(15)].get(mode="promise_in_bounds")
            return tot

        return lax.fori_loop(0, _XCHUNK // 64, grp, n)

    n_v = lax.fori_loop(
        0, _B // _XCHUNK, chunk_scan, jnp.zeros((16,), jnp.int32)
    )
    n = jnp.max(n_v)
    for u in range(4):
        plsc.store_scatter(lb, [n_v + u * 16 + lanes], _i16(-1))
    ng = (n + 63) >> 6

    def process(j, q, nscat, xshift=0):
        def fill(w):
            wlo = w * _CAP

            def fill_grp(g, tot):
                for u in range(4):
                    sl = pl.ds(g * 64 + u * 16, 16)
                    m = lb[sl] == j
                    cs = plsc.cumsum(jnp.where(m, 1, 0))
                    slots = cs + tot - 1
                    mw = m & (slots >= wlo) & (slots < wlo + _CAP)
                    plsc.store_scatter(
                        blx, [slots - wlo], lx[sl] - (j * _BLK - xshift), mask=mw
                    )
                    plsc.store_scatter(blp, [slots - wlo], lp[sl], mask=mw)
                    tot = tot + cs.at[_i16(15)].get(mode="promise_in_bounds")
                return tot

            return jnp.max(
                lax.fori_loop(0, ng, fill_grp, jnp.zeros((16,), jnp.int32))
            )

        def extract(kw, nscat):
            def grp(g, nscat):
                slot = (nscat >> 2) & 1
                rbase = (nscat & 3) * 16

                @pl.when((nscat & 3) == 0)
                def _():
                    @pl.when((nscat >> 2) >= 2)
                    def _():
                        pltpu.make_async_copy(
                            out_hbm.at[pl.ds(0, 64)], stage.at[0], sem_out
                        ).wait()

                    for u in range(4):
                        sidx[slot, pl.ds(u * 16, 16)] = _TRASH + lanes

                valid = lanes < (kw - g * 16)
                xloc = jnp.where(valid, blx[pl.ds(g * 16, 16)], 0)
                pos = jnp.where(valid, blp[pl.ds(g * 16, 16)], _TRASH + lanes)
                qv = jnp.full((16,), q, dtype=jnp.int32)
                acc = [jnp.full((16,), -jnp.inf, dtype=jnp.float32)] * 4
                for d in range(_D):
                    c = plsc.load_gather(bufs, [qv, _i16(d), xloc])
                    colb[d, :] = c
                    acc[d % 4] = jnp.maximum(acc[d % 4], c)
                m = jnp.maximum(
                    jnp.maximum(acc[0], acc[1]), jnp.maximum(acc[2], acc[3])
                )
                sacc = [jnp.zeros((16,), jnp.float32)] * 4
                for d in range(_D):
                    e = jnp.exp(colb[d, :] - m)
                    colb[d, :] = e
                    sacc[d % 4] = sacc[d % 4] + e
                s = (sacc[0] + sacc[1]) + (sacc[2] + sacc[3])
                inv = 1.0 / s
                for d in range(_D):
                    plsc.store_scatter(
                        stage.at[slot], [rbase + lanes, _i16(d)], colb[d, :] * inv
                    )
                sidx[slot, pl.ds(rbase, 16)] = pos

                @pl.when((nscat & 3) == 3)
                def _():
                    pltpu.async_copy(
                        stage.at[slot], out_hbm.at[sidx.at[slot]], sem_out
                    )

                return nscat + 1

            return lax.fori_loop(0, (kw + 15) >> 4, grp, nscat)

        ktot = fill(jnp.int32(0))
        nscat = extract(jnp.minimum(ktot, _CAP), nscat)
        nwaves = (ktot + _CAP - 1) >> 8

        def wave(w, nscat):
            k_end = fill(w)
            return extract(jnp.minimum(k_end - w * _CAP, _CAP), nscat)

        return lax.fori_loop(1, nwaves, wave, nscat)

    def fire(j, sem):
        col0 = pl.multiple_of(lo + j * _BLK, 128)
        pltpu.async_copy(
            tt_hbm.at[:, pl.ds(col0, _BLK)], bufs.at[j % 2], sem
        )

    def wait_in(j, sem):
        pltpu.make_async_copy(
            tt_hbm.at[:, pl.ds(0, _BLK)], bufs.at[j % 2], sem
        ).wait()

    fire(0, sem0)

    def blk_body(j, nscat):
        @pl.when(j + 1 < nblk)
        def _():
            @pl.when((j + 1) % 2 == 0)
            def _():
                fire(j + 1, sem0)

            @pl.when((j + 1) % 2 == 1)
            def _():
                fire(j + 1, sem1)

        @pl.when(j % 2 == 0)
        def _():
            wait_in(j, sem0)

        @pl.when(j % 2 == 1)
        def _():
            wait_in(j, sem1)

        return process(j, j % 2, nscat)

    nscat = lax.fori_loop(0, nblk, blk_body, jnp.int32(0))

    def tail(nscat):
        pltpu.sync_copy(tail_hbm, bufs.at[0, :, pl.ds(0, 128)])
        return process(nblk, jnp.int32(0), nscat)

    nscat = lax.cond(is_last, tail, lambda ns: ns, nscat)

    @pl.when((nscat & 3) != 0)
    def _():
        slot = (nscat >> 2) & 1
        pltpu.async_copy(stage.at[slot], out_hbm.at[sidx.at[slot]], sem_out)

    def drain(i, _):
        pltpu.make_async_copy(
            out_hbm.at[pl.ds(0, 64)], stage.at[0], sem_out
        ).wait()
        return _

    lax.fori_loop(0, jnp.minimum((nscat + 3) >> 2, 2), drain, None)


def kernel(x, table):
    tt = table.T
    tail = jnp.pad(tt[:, _FULL_BLKS * _BLK :], ((0, 0), (0, _BLK // 2 - _TAIL_W)))
    out = _decoder_kernel(x.astype(jnp.int32), tt, tail)
    return out[:_B, :_D]
